# Initial kernel scaffold; baseline (speedup 1.0000x reference)
import jax
import jax.numpy as jnp
from jax.experimental import pallas as pl


def kernel(tokens, table):
    raise NotImplementedError("write your pallas kernel here")


# speedup vs baseline: 2.2898x; 2.2898x over previous
import functools

import jax
import jax.numpy as jnp
from jax import lax
from jax.experimental import pallas as pl
from jax.experimental.pallas import tpu as pltpu
from jax.experimental.pallas import tpu_sc as plsc

_PAD = 1
_NUM_CORES = 2
_NUM_SUBCORES = 16
_NUM_WORKERS = _NUM_CORES * _NUM_SUBCORES


def _positions_body(tok_ref, pos_ref):
    t = tok_ref[...]
    m = (t != _PAD).astype(jnp.int32)
    x = m
    n = t.shape[1]
    k = 1
    while k < n:
        shifted = jnp.concatenate(
            [jnp.zeros((t.shape[0], k), jnp.int32), x[:, :-k]], axis=1
        )
        x = x + shifted
        k *= 2
    pos_ref[...] = x * m + _PAD


def _compute_positions(tokens):
    return pl.pallas_call(
        _positions_body,
        out_shape=jax.ShapeDtypeStruct(tokens.shape, jnp.int32),
    )(tokens)


@functools.lru_cache(maxsize=None)
def _make_gather(n_rows, dim, chunk):
    rows_per_w = n_rows // _NUM_WORKERS
    nchunk = rows_per_w // chunk
    mesh = plsc.VectorSubcoreMesh(core_axis_name="c", subcore_axis_name="s")

    @functools.partial(
        pl.kernel,
        mesh=mesh,
        out_type=jax.ShapeDtypeStruct((n_rows, dim), jnp.float32),
        scratch_types=[
            pltpu.VMEM((nchunk, chunk), jnp.int32),
            pltpu.VMEM((chunk, dim), jnp.float32),
            pltpu.VMEM((chunk, dim), jnp.float32),
            pltpu.SemaphoreType.DMA,
            pltpu.SemaphoreType.DMA,
        ],
    )
    def gather_kernel(table_hbm, idx_hbm, out_hbm, idx_v, buf0, buf1, sem0, sem1):
        wid = lax.axis_index("s") * _NUM_CORES + lax.axis_index("c")
        base = wid * rows_per_w
        pltpu.sync_copy(idx_hbm.at[wid], idx_v)

        bufs = (buf0, buf1)
        sems = (sem0, sem1)

        def start(g, b):
            pltpu.make_async_copy(table_hbm.at[idx_v.at[g]], bufs[b], sems[b]).start()

        def wait(b):
            pltpu.make_async_copy(table_hbm.at[idx_v.at[0]], bufs[b], sems[b]).wait()

        def write(g, b):
            pltpu.sync_copy(bufs[b], out_hbm.at[pl.ds(base + g * chunk, chunk)])

        start(0, 0)
        start(1, 1)

        @pl.loop(0, nchunk - 2, step=2)
        def _(g):
            wait(0)
            write(g, 0)
            start(g + 2, 0)
            wait(1)
            write(g + 1, 1)
            start(g + 3, 1)

        g_last = nchunk - 2
        wait(0)
        write(g_last, 0)
        wait(1)
        write(g_last + 1, 1)

    return gather_kernel


def kernel(tokens, table):
    batch, seq = tokens.shape
    n_rows = batch * seq
    dim = table.shape[1]
    chunk = 32

    positions = _compute_positions(tokens)
    idx = positions.reshape(_NUM_WORKERS, (n_rows // _NUM_WORKERS) // chunk, chunk)
    out = _make_gather(n_rows, dim, chunk)(table, idx)
    return out.reshape(batch, seq, dim)
